# baseline (device time: 16421 ns/iter reference)
import jax
import jax.numpy as jnp
from jax import lax
from jax.experimental import pallas as pl
from jax.experimental.pallas import tpu as pltpu

N_DEV = 16
N_PLANE = 4
GRP = N_DEV // N_PLANE
N_CHUNK = 8
BLK_PER_CHUNK = 2

_PLANE_ORDER = {
    0: (3, 2, 1, 0),
    1: (3, 2, 0, 1),
    2: (0, 1, 3, 2),
    3: (0, 1, 2, 3),
}


def _plane_at(my_grp, t):
    return jnp.where(
        my_grp == 0, _PLANE_ORDER[0][t],
        jnp.where(my_grp == 1, _PLANE_ORDER[1][t],
                  jnp.where(my_grp == 2, _PLANE_ORDER[2][t],
                            _PLANE_ORDER[3][t])),
    )


def kernel(x, w_mat):
    m_per, k = x.shape
    _, n = w_mat.shape
    n_per = n // N_DEV
    n_chunk = n // N_CHUNK

    def body(x_hbm, w_hbm, out_hbm, xbuf, wbuf, ybuf, rbuf, obuf,
             xsem, wsems, osem, send_sems, recv_sems):
        me = lax.axis_index("i")
        my_grp = me // GRP
        my_lane = me % GRP

        xcp = pltpu.make_async_copy(x_hbm, xbuf, xsem)
        xcp.start()
        wcps = []
        for c in range(N_CHUNK):
            g = _plane_at(my_grp, c // 2)
            half = c % 2
            cp = pltpu.make_async_copy(
                w_hbm.at[:, pl.ds(g * (2 * n_chunk) + half * n_chunk, n_chunk)],
                wbuf.at[c], wsems.at[c],
            )
            cp.start()
            wcps.append(cp)

        barrier = pltpu.get_barrier_semaphore()
        for s in range(1, N_DEV):
            pl.semaphore_signal(
                barrier, inc=1,
                device_id=((me + s) % N_DEV,),
                device_id_type=pl.DeviceIdType.MESH,
            )
        pl.semaphore_wait(barrier, N_DEV - 1)

        xcp.wait()
        x_val = xbuf[:, :]

        for c in range(N_CHUNK):
            g = _plane_at(my_grp, c // 2)
            half = c % 2
            wcps[c].wait()
            y_val = jnp.dot(
                x_val, wbuf[c, :, :], preferred_element_type=jnp.float32,
            )
            ybuf[c, :, :] = y_val.astype(jnp.bfloat16)
            for b in range(BLK_PER_CHUNK):
                lane = half * BLK_PER_CHUNK + b
                d = g * GRP + lane
                if c >= N_CHUNK - 2:
                    @pl.when(lane == my_lane)
                    def _():
                        obuf[pl.ds(me * m_per, m_per), :] = (
                            y_val[:, b * n_per:(b + 1) * n_per]
                        )

                    @pl.when(lane != my_lane)
                    def _():
                        rdma = pltpu.make_async_remote_copy(
                            src_ref=ybuf.at[c, :, pl.ds(b * n_per, n_per)],
                            dst_ref=rbuf.at[me],
                            send_sem=send_sems.at[d],
                            recv_sem=recv_sems.at[me],
                            device_id=(d,),
                            device_id_type=pl.DeviceIdType.MESH,
                        )
                        rdma.start()
                else:
                    rdma = pltpu.make_async_remote_copy(
                        src_ref=ybuf.at[c, :, pl.ds(b * n_per, n_per)],
                        dst_ref=rbuf.at[me],
                        send_sem=send_sems.at[d],
                        recv_sem=recv_sems.at[me],
                        device_id=(d,),
                        device_id_type=pl.DeviceIdType.MESH,
                    )
                    rdma.start()

        for t in range(N_PLANE):
            h = _plane_at(my_grp, t)
            for b in range(GRP):
                j = h * GRP + b

                def _wait_and_store(j=j):
                    recv = pltpu.make_async_remote_copy(
                        src_ref=ybuf.at[0, :, pl.ds(0, n_per)],
                        dst_ref=rbuf.at[j],
                        send_sem=send_sems.at[j],
                        recv_sem=recv_sems.at[j],
                        device_id=(me,),
                        device_id_type=pl.DeviceIdType.MESH,
                    )
                    recv.wait_recv()
                    obuf[pl.ds(j * m_per, m_per), :] = (
                        rbuf[j, :, :].astype(jnp.float32)
                    )

                if t == N_PLANE - 1:
                    pl.when(b != my_lane)(_wait_and_store)
                else:
                    _wait_and_store()

        ocp = pltpu.make_async_copy(obuf, out_hbm, osem)
        ocp.start()

        for d in range(N_DEV):
            @pl.when(d != me)
            def _():
                snd = pltpu.make_async_remote_copy(
                    src_ref=ybuf.at[0, :, pl.ds(0, n_per)],
                    dst_ref=rbuf.at[0],
                    send_sem=send_sems.at[d],
                    recv_sem=recv_sems.at[d],
                    device_id=(me,),
                    device_id_type=pl.DeviceIdType.MESH,
                )
                snd.wait_send()

        ocp.wait()

    x = pltpu.with_memory_space_constraint(x, pltpu.MemorySpace.HBM)
    w_mat = pltpu.with_memory_space_constraint(w_mat, pltpu.MemorySpace.HBM)
    return pl.pallas_call(
        body,
        out_shape=jax.ShapeDtypeStruct((N_DEV * m_per, n_per), jnp.float32),
        in_specs=[
            pl.BlockSpec(memory_space=pltpu.MemorySpace.HBM),
            pl.BlockSpec(memory_space=pltpu.MemorySpace.HBM),
        ],
        out_specs=pl.BlockSpec(memory_space=pltpu.MemorySpace.HBM),
        scratch_shapes=[
            pltpu.VMEM((m_per, k), jnp.float32),
            pltpu.VMEM((N_CHUNK, k, n_chunk), jnp.float32),
            pltpu.VMEM((N_CHUNK, m_per, n_chunk), jnp.bfloat16),
            pltpu.VMEM((N_DEV, m_per, n_per), jnp.bfloat16),
            pltpu.VMEM((N_DEV * m_per, n_per), jnp.float32),
            pltpu.SemaphoreType.DMA,
            pltpu.SemaphoreType.DMA((N_CHUNK,)),
            pltpu.SemaphoreType.DMA,
            pltpu.SemaphoreType.DMA((N_DEV,)),
            pltpu.SemaphoreType.DMA((N_DEV,)),
        ],
        compiler_params=pltpu.CompilerParams(collective_id=0),
    )(x, w_mat)
